# baseline (speedup 1.0000x reference)
import functools

import jax
import jax.numpy as jnp
from jax import lax
from jax.experimental import pallas as pl
from jax.experimental.pallas import tpu as pltpu
from jax.experimental.pallas import tpu_sc as plsc

B = 16384
L = 200
NC = 2
NS = 16
NW = NC * NS
BPW = B // NW
VECS = BPW // 16
GCHUNK = 128
NGATHER = BPW // GCHUNK


def _sc_kernel(node_hbm, items_hbm, out_hbm, node_v, idx_v, val_v, sem):
    wid = lax.axis_index("s") * NC + lax.axis_index("c")
    base = wid * BPW
    pltpu.sync_copy(node_hbm.at[pl.ds(base, BPW)], node_v)

    lanes = lax.iota(jnp.int32, 16)
    for i in range(VECS):
        node = node_v[pl.ds(i * 16, 16)]
        rows = (base + i * 16) + lanes
        nm1 = jnp.clip(node - 1, 0, L - 1)
        idx_v[i // (GCHUNK // 16), pl.ds((i % (GCHUNK // 16)) * 16, 16)] = rows * L + nm1

    for j in range(NGATHER):
        pltpu.async_copy(
            items_hbm.at[idx_v.at[j]],
            val_v.at[pl.ds(j * GCHUNK, GCHUNK)],
            sem,
        ).wait()

    for i in range(VECS):
        node = node_v[pl.ds(i * 16, 16)]
        val = val_v[pl.ds(i * 16, 16)]
        val_v[pl.ds(i * 16, 16)] = jnp.where(node != 0, val, jnp.int32(-1))

    pltpu.sync_copy(val_v, out_hbm.at[pl.ds(base, BPW)])


@jax.jit
def _decode(node, items_flat):
    mesh = plsc.VectorSubcoreMesh(core_axis_name="c", subcore_axis_name="s")
    run = functools.partial(
        pl.kernel,
        mesh=mesh,
        out_type=jax.ShapeDtypeStruct((B,), jnp.int32),
        scratch_types=[
            pltpu.VMEM((BPW,), jnp.int32),
            pltpu.VMEM((NGATHER, GCHUNK), jnp.int32),
            pltpu.VMEM((BPW,), jnp.int32),
            pltpu.SemaphoreType.DMA,
        ],
    )(_sc_kernel)
    return run(node, items_flat)


def kernel(x_dummy, current_node, item_ids):
    node = jnp.reshape(current_node, (B,)).astype(jnp.int32)
    items_flat = jnp.reshape(item_ids, (B * L,)).astype(jnp.int32)
    return _decode(node, items_flat).astype(item_ids.dtype)


# speedup vs baseline: 1.4759x; 1.4759x over previous
import functools

import jax
import jax.numpy as jnp
from jax import lax
from jax.experimental import pallas as pl
from jax.experimental.pallas import tpu as pltpu
from jax.experimental.pallas import tpu_sc as plsc

B = 16384
L = 200
NC = 2
NS = 16
NW = NC * NS
BPW = B // NW
ROWCHUNK = 128
NCHUNK = BPW // ROWCHUNK


def _sc_kernel(node_hbm, items_hbm, out_hbm, node_v, rows_v, out_v, sem):
    wid = lax.axis_index("s") * NC + lax.axis_index("c")
    base = wid * BPW
    pltpu.sync_copy(node_hbm.at[pl.ds(base, BPW)], node_v)

    lanes = lax.iota(jnp.int32, 16)
    for c in range(NCHUNK):
        pltpu.sync_copy(items_hbm.at[pl.ds(base + c * ROWCHUNK, ROWCHUNK), :],
                        rows_v)
        for i in range(ROWCHUNK // 16):
            v = c * (ROWCHUNK // 16) + i
            node = node_v[pl.ds(v * 16, 16)]
            nm1 = jnp.clip(node - 1, 0, L - 1)
            val = plsc.load_gather(rows_v, [(i * 16) + lanes, nm1])
            out_v[pl.ds(v * 16, 16)] = jnp.where(node != 0, val, jnp.int32(-1))

    pltpu.sync_copy(out_v, out_hbm.at[pl.ds(base, BPW)])


@jax.jit
def _decode(node, items):
    mesh = plsc.VectorSubcoreMesh(core_axis_name="c", subcore_axis_name="s")
    run = functools.partial(
        pl.kernel,
        mesh=mesh,
        out_type=jax.ShapeDtypeStruct((B,), jnp.int32),
        scratch_types=[
            pltpu.VMEM((BPW,), jnp.int32),
            pltpu.VMEM((ROWCHUNK, L), jnp.int32),
            pltpu.VMEM((BPW,), jnp.int32),
            pltpu.SemaphoreType.DMA,
        ],
        compiler_params=pltpu.CompilerParams(
            use_tc_tiling_on_sc=True, needs_layout_passes=False),
    )(_sc_kernel)
    return run(node, items)


def kernel(x_dummy, current_node, item_ids):
    node = jnp.reshape(current_node, (B,)).astype(jnp.int32)
    return _decode(node, item_ids.astype(jnp.int32)).astype(item_ids.dtype)
